# baseline (speedup 1.0000x reference)
import functools

import jax
import jax.numpy as jnp
from jax import lax
from jax.experimental import pallas as pl
from jax.experimental.pallas import tpu as pltpu
from jax.experimental.pallas import tpu_sc as plsc

_D = 768
_TB = 256
_KB = 512
_COMMIT_W = 0.25


def _dist_body_first(a2_ref, r_ref, cb_ref, b2_ref, idx_ref,
                     minv_s, mina_s):
    j = pl.program_id(1)
    kt = pl.num_programs(1)
    r = r_ref[...]
    _dist_common(r, a2_ref[0, 0, :], cb_ref, b2_ref, idx_ref, minv_s, mina_s,
                 j, kt)


def _dist_body_update(r_ref, q_ref, cb_ref, b2_ref, idx_ref, rout_ref,
                      a2out_ref, rn_s, a2_s, minv_s, mina_s):
    j = pl.program_id(1)
    kt = pl.num_programs(1)

    @pl.when(j == 0)
    def _():
        r = r_ref[...]
        q = q_ref[...]
        qst = r + (q - r)
        rn = r - qst
        rn_s[...] = rn
        rout_ref[...] = rn
        a2row = jnp.sum(rn * rn, axis=1)
        a2_s[0, :] = a2row
        a2out_ref[0, 0, :] = a2row

    rn = rn_s[...]
    _dist_common(rn, a2_s[0, :], cb_ref, b2_ref, idx_ref, minv_s, mina_s,
                 j, kt)


def _dist_common(r, a2, cb_ref, b2_ref, idx_ref, minv_s, mina_s, j, kt):
    cb = cb_ref[...]
    ab = lax.dot_general(r, cb, (((1,), (1,)), ((), ())))
    d2 = jnp.maximum((a2[:, None] + b2_ref[...]) - 2.0 * ab, 0.0)
    loc_min = jnp.min(d2, axis=1, keepdims=True)
    ii = lax.broadcasted_iota(jnp.int32, d2.shape, 1)
    big = jnp.int32(2 ** 30)
    loc_arg = jnp.min(jnp.where(d2 == loc_min, ii, big), axis=1,
                      keepdims=True) + j * _KB

    @pl.when(j == 0)
    def _():
        minv_s[...] = loc_min
        mina_s[...] = loc_arg

    @pl.when(j != 0)
    def _():
        better = loc_min < minv_s[...]
        mina_s[...] = jnp.where(better, loc_arg, mina_s[...])
        minv_s[...] = jnp.where(better, loc_min, minv_s[...])

    @pl.when(j == kt - 1)
    def _():
        idx_ref[0, 0, :] = mina_s[:, 0]


def _dist_first(xf, a2, cb, b2, nb, k):
    kt = k // _KB
    return pl.pallas_call(
        _dist_body_first,
        grid=(nb, kt),
        in_specs=[
            pl.BlockSpec((1, 1, _TB), lambda i, j: (i, 0, 0)),
            pl.BlockSpec((_TB, _D), lambda i, j: (i, 0)),
            pl.BlockSpec((_KB, _D), lambda i, j: (j, 0)),
            pl.BlockSpec((1, _KB), lambda i, j: (0, j)),
        ],
        out_specs=pl.BlockSpec((1, 1, _TB), lambda i, j: (i, 0, 0)),
        out_shape=jax.ShapeDtypeStruct((nb, 1, _TB), jnp.int32),
        scratch_shapes=[
            pltpu.VMEM((_TB, 1), jnp.float32),
            pltpu.VMEM((_TB, 1), jnp.int32),
        ],
    )(a2, xf, cb, b2)


def _dist_update(r, q, cb, b2, nb, k):
    kt = k // _KB
    n = nb * _TB
    return pl.pallas_call(
        _dist_body_update,
        grid=(nb, kt),
        in_specs=[
            pl.BlockSpec((_TB, _D), lambda i, j: (i, 0)),
            pl.BlockSpec((_TB, _D), lambda i, j: (i, 0)),
            pl.BlockSpec((_KB, _D), lambda i, j: (j, 0)),
            pl.BlockSpec((1, _KB), lambda i, j: (0, j)),
        ],
        out_specs=[
            pl.BlockSpec((1, 1, _TB), lambda i, j: (i, 0, 0)),
            pl.BlockSpec((_TB, _D), lambda i, j: (i, 0)),
            pl.BlockSpec((1, 1, _TB), lambda i, j: (i, 0, 0)),
        ],
        out_shape=[
            jax.ShapeDtypeStruct((nb, 1, _TB), jnp.int32),
            jax.ShapeDtypeStruct((n, _D), jnp.float32),
            jax.ShapeDtypeStruct((nb, 1, _TB), jnp.float32),
        ],
        scratch_shapes=[
            pltpu.VMEM((_TB, _D), jnp.float32),
            pltpu.VMEM((1, _TB), jnp.float32),
            pltpu.VMEM((_TB, 1), jnp.float32),
            pltpu.VMEM((_TB, 1), jnp.int32),
        ],
    )(r, q, cb, b2)


def _final_body(x_ref, r_ref, q_ref, quant_ref, a2out_ref):
    x = x_ref[...]
    r = r_ref[...]
    q = q_ref[...]
    qst = r + (q - r)
    rn = r - qst
    quant_ref[...] = x - rn
    a2out_ref[0, 0, :] = jnp.sum(rn * rn, axis=1)


def _final(xf, r, q, nb):
    n = nb * _TB
    return pl.pallas_call(
        _final_body,
        grid=(nb,),
        in_specs=[
            pl.BlockSpec((_TB, _D), lambda i: (i, 0)),
            pl.BlockSpec((_TB, _D), lambda i: (i, 0)),
            pl.BlockSpec((_TB, _D), lambda i: (i, 0)),
        ],
        out_specs=[
            pl.BlockSpec((_TB, _D), lambda i: (i, 0)),
            pl.BlockSpec((1, 1, _TB), lambda i: (i, 0, 0)),
        ],
        out_shape=[
            jax.ShapeDtypeStruct((n, _D), jnp.float32),
            jax.ShapeDtypeStruct((nb, 1, _TB), jnp.float32),
        ],
    )(xf, r, q)


def _make_sc_gather(n_tokens):
    info = plsc.get_sparse_core_info()
    nw = info.num_cores * info.num_subcores
    bpw = n_tokens // nw
    mesh = plsc.VectorSubcoreMesh(core_axis_name="c", subcore_axis_name="s")

    def body(table_hbm, idx_hbm, out_hbm, idx_v, rows_v, sem):
        wid = lax.axis_index("s") * info.num_cores + lax.axis_index("c")
        base = wid * bpw
        pltpu.sync_copy(idx_hbm.at[pl.ds(base, bpw)], idx_v)
        pltpu.async_copy(table_hbm.at[idx_v], rows_v, sem).wait()
        pltpu.sync_copy(rows_v, out_hbm.at[pl.ds(base, bpw)])

    return functools.partial(
        pl.kernel,
        mesh=mesh,
        out_type=jax.ShapeDtypeStruct((n_tokens, _D), jnp.float32),
        scratch_types=[
            pltpu.VMEM((bpw,), jnp.int32),
            pltpu.VMEM((bpw, _D), jnp.float32),
            pltpu.SemaphoreType.DMA,
        ],
    )(body)


def kernel(x, codebook_0, codebook_1, codebook_2, codebook_3):
    codebooks = [codebook_0, codebook_1, codebook_2, codebook_3]
    b, t, d = x.shape
    n = b * t
    nb = n // _TB
    xf = x.reshape(n, d)

    b2s = [jnp.sum(cb * cb, axis=1)[None, :] for cb in codebooks]
    a2_0 = jnp.sum(xf * xf, axis=1).reshape(nb, 1, _TB)

    sc_gather = _make_sc_gather(n)

    idx0 = _dist_first(xf, a2_0, codebooks[0], b2s[0], nb, codebooks[0].shape[0])
    q = sc_gather(codebooks[0], idx0.reshape(n))

    indices = [idx0]
    a2_sums = []
    r = xf
    for l in (1, 2, 3):
        k = codebooks[l].shape[0]
        idx_l, r, a2_l = _dist_update(r, q, codebooks[l], b2s[l], nb, k)
        indices.append(idx_l)
        a2_sums.append(jnp.sum(a2_l))
        q = sc_gather(codebooks[l], idx_l.reshape(n))

    quant, a2_last = _final(xf, r, q, nb)
    a2_sums.append(jnp.sum(a2_last))

    total_commit = jnp.asarray(0.0, dtype=jnp.float32)
    scale = jnp.float32(_COMMIT_W / (n * d))
    for s in a2_sums:
        total_commit = total_commit + s * scale

    quantized = quant.reshape(b, t, d)
    all_indices = jnp.stack([ix.reshape(b, t) for ix in indices], axis=-1)
    return quantized, all_indices, total_commit


# speedup vs baseline: 1.0019x; 1.0019x over previous
import functools

import jax
import jax.numpy as jnp
from jax import lax
from jax.experimental import pallas as pl
from jax.experimental.pallas import tpu as pltpu
from jax.experimental.pallas import tpu_sc as plsc

_D = 768
_TB = 256
_KB = 512
_COMMIT_W = 0.25


def _dist_common(rT, a2v, cb_ref, b2_ref, idx_ref, minv_s, mina_s, j, kt):
    cb = cb_ref[...]
    ab = lax.dot_general(cb, rT, (((1,), (0,)), ((), ())))
    d2 = jnp.maximum((b2_ref[...] + a2v) - 2.0 * ab, 0.0)
    loc_min = jnp.min(d2, axis=0, keepdims=True)
    ii = lax.broadcasted_iota(jnp.int32, d2.shape, 0)
    big = jnp.int32(2 ** 30)
    loc_arg = jnp.min(jnp.where(d2 == loc_min, ii, big), axis=0,
                      keepdims=True) + j * _KB

    @pl.when(j == 0)
    def _():
        minv_s[...] = loc_min
        mina_s[...] = loc_arg

    @pl.when(j != 0)
    def _():
        better = loc_min < minv_s[...]
        mina_s[...] = jnp.where(better, loc_arg, mina_s[...])
        minv_s[...] = jnp.where(better, loc_min, minv_s[...])

    @pl.when(j == kt - 1)
    def _():
        idx_ref[0, 0, :] = mina_s[0, :]


def _dist_body_first(a2_ref, r_ref, cb_ref, b2_ref, idx_ref,
                     rT_s, minv_s, mina_s):
    j = pl.program_id(1)
    kt = pl.num_programs(1)

    @pl.when(j == 0)
    def _():
        rT_s[...] = r_ref[...].T

    _dist_common(rT_s[...], a2_ref[0], cb_ref, b2_ref, idx_ref,
                 minv_s, mina_s, j, kt)


def _dist_body_update(r_ref, q_ref, cb_ref, b2_ref, idx_ref, rout_ref,
                      a2out_ref, rT_s, a2_s, minv_s, mina_s):
    j = pl.program_id(1)
    kt = pl.num_programs(1)

    @pl.when(j == 0)
    def _():
        r = r_ref[...]
        q = q_ref[...]
        qst = r + (q - r)
        rn = r - qst
        rout_ref[...] = rn
        rT_s[...] = rn.T
        a2row = jnp.sum(rn * rn, axis=1)
        a2_s[0, :] = a2row
        a2out_ref[0, 0, :] = a2row

    _dist_common(rT_s[...], a2_s[...], cb_ref, b2_ref, idx_ref,
                 minv_s, mina_s, j, kt)


def _dist_first(xf, a2, cb, b2, nb, k):
    kt = k // _KB
    return pl.pallas_call(
        _dist_body_first,
        grid=(nb, kt),
        in_specs=[
            pl.BlockSpec((1, 1, _TB), lambda i, j: (i, 0, 0)),
            pl.BlockSpec((_TB, _D), lambda i, j: (i, 0)),
            pl.BlockSpec((_KB, _D), lambda i, j: (j, 0)),
            pl.BlockSpec((_KB, 1), lambda i, j: (j, 0)),
        ],
        out_specs=pl.BlockSpec((1, 1, _TB), lambda i, j: (i, 0, 0)),
        out_shape=jax.ShapeDtypeStruct((nb, 1, _TB), jnp.int32),
        scratch_shapes=[
            pltpu.VMEM((_D, _TB), jnp.float32),
            pltpu.VMEM((1, _TB), jnp.float32),
            pltpu.VMEM((1, _TB), jnp.int32),
        ],
    )(a2, xf, cb, b2)


def _dist_update(r, q, cb, b2, nb, k):
    kt = k // _KB
    n = nb * _TB
    return pl.pallas_call(
        _dist_body_update,
        grid=(nb, kt),
        in_specs=[
            pl.BlockSpec((_TB, _D), lambda i, j: (i, 0)),
            pl.BlockSpec((_TB, _D), lambda i, j: (i, 0)),
            pl.BlockSpec((_KB, _D), lambda i, j: (j, 0)),
            pl.BlockSpec((_KB, 1), lambda i, j: (j, 0)),
        ],
        out_specs=[
            pl.BlockSpec((1, 1, _TB), lambda i, j: (i, 0, 0)),
            pl.BlockSpec((_TB, _D), lambda i, j: (i, 0)),
            pl.BlockSpec((1, 1, _TB), lambda i, j: (i, 0, 0)),
        ],
        out_shape=[
            jax.ShapeDtypeStruct((nb, 1, _TB), jnp.int32),
            jax.ShapeDtypeStruct((n, _D), jnp.float32),
            jax.ShapeDtypeStruct((nb, 1, _TB), jnp.float32),
        ],
        scratch_shapes=[
            pltpu.VMEM((_D, _TB), jnp.float32),
            pltpu.VMEM((1, _TB), jnp.float32),
            pltpu.VMEM((1, _TB), jnp.float32),
            pltpu.VMEM((1, _TB), jnp.int32),
        ],
    )(r, q, cb, b2)


def _final_body(x_ref, r_ref, q_ref, quant_ref, a2out_ref):
    x = x_ref[...]
    r = r_ref[...]
    q = q_ref[...]
    qst = r + (q - r)
    rn = r - qst
    quant_ref[...] = x - rn
    a2out_ref[0, 0, :] = jnp.sum(rn * rn, axis=1)


def _final(xf, r, q, nb):
    n = nb * _TB
    return pl.pallas_call(
        _final_body,
        grid=(nb,),
        in_specs=[
            pl.BlockSpec((_TB, _D), lambda i: (i, 0)),
            pl.BlockSpec((_TB, _D), lambda i: (i, 0)),
            pl.BlockSpec((_TB, _D), lambda i: (i, 0)),
        ],
        out_specs=[
            pl.BlockSpec((_TB, _D), lambda i: (i, 0)),
            pl.BlockSpec((1, 1, _TB), lambda i: (i, 0, 0)),
        ],
        out_shape=[
            jax.ShapeDtypeStruct((n, _D), jnp.float32),
            jax.ShapeDtypeStruct((nb, 1, _TB), jnp.float32),
        ],
    )(xf, r, q)


def _make_sc_gather(n_tokens):
    info = plsc.get_sparse_core_info()
    nw = info.num_cores * info.num_subcores
    bpw = n_tokens // nw
    mesh = plsc.VectorSubcoreMesh(core_axis_name="c", subcore_axis_name="s")

    def body(table_hbm, idx_hbm, out_hbm, idx_v, rows_v, sem):
        wid = lax.axis_index("s") * info.num_cores + lax.axis_index("c")
        base = wid * bpw
        pltpu.sync_copy(idx_hbm.at[pl.ds(base, bpw)], idx_v)
        pltpu.async_copy(table_hbm.at[idx_v], rows_v, sem).wait()
        pltpu.sync_copy(rows_v, out_hbm.at[pl.ds(base, bpw)])

    return functools.partial(
        pl.kernel,
        mesh=mesh,
        out_type=jax.ShapeDtypeStruct((n_tokens, _D), jnp.float32),
        scratch_types=[
            pltpu.VMEM((bpw,), jnp.int32),
            pltpu.VMEM((bpw, _D), jnp.float32),
            pltpu.SemaphoreType.DMA,
        ],
    )(body)


def kernel(x, codebook_0, codebook_1, codebook_2, codebook_3):
    codebooks = [codebook_0, codebook_1, codebook_2, codebook_3]
    b, t, d = x.shape
    n = b * t
    nb = n // _TB
    xf = x.reshape(n, d)

    b2s = [jnp.sum(cb * cb, axis=1)[:, None] for cb in codebooks]
    a2_0 = jnp.sum(xf * xf, axis=1).reshape(nb, 1, _TB)

    sc_gather = _make_sc_gather(n)

    idx0 = _dist_first(xf, a2_0, codebooks[0], b2s[0], nb, codebooks[0].shape[0])
    q = sc_gather(codebooks[0], idx0.reshape(n))

    indices = [idx0]
    a2_sums = []
    r = xf
    for l in (1, 2, 3):
        k = codebooks[l].shape[0]
        idx_l, r, a2_l = _dist_update(r, q, codebooks[l], b2s[l], nb, k)
        indices.append(idx_l)
        a2_sums.append(jnp.sum(a2_l))
        q = sc_gather(codebooks[l], idx_l.reshape(n))

    quant, a2_last = _final(xf, r, q, nb)
    a2_sums.append(jnp.sum(a2_last))

    total_commit = jnp.asarray(0.0, dtype=jnp.float32)
    scale = jnp.float32(_COMMIT_W / (n * d))
    for s in a2_sums:
        total_commit = total_commit + s * scale

    quantized = quant.reshape(b, t, d)
    all_indices = jnp.stack([ix.reshape(b, t) for ix in indices], axis=-1)
    return quantized, all_indices, total_commit


# speedup vs baseline: 1.2597x; 1.2573x over previous
import functools

import jax
import jax.numpy as jnp
from jax import lax
from jax.experimental import pallas as pl
from jax.experimental.pallas import tpu as pltpu
from jax.experimental.pallas import tpu_sc as plsc

_D = 768
_TB = 256
_KB = 1024
_COMMIT_W = 0.25


def _dist_common(rT, a2v, cb_ref, b2_ref, idx_ref, minv_s, mina_s, j, kt):
    cb = cb_ref[...]
    ab = lax.dot_general(cb, rT, (((1,), (0,)), ((), ())))
    d2 = jnp.maximum((b2_ref[...] + a2v) - 2.0 * ab, 0.0)
    loc_min = jnp.min(d2, axis=0, keepdims=True)
    ii = lax.broadcasted_iota(jnp.int32, d2.shape, 0)
    big = jnp.int32(2 ** 30)
    loc_arg = jnp.min(jnp.where(d2 == loc_min, ii, big), axis=0,
                      keepdims=True) + j * _KB

    @pl.when(j == 0)
    def _():
        minv_s[...] = loc_min
        mina_s[...] = loc_arg

    @pl.when(j != 0)
    def _():
        better = loc_min < minv_s[...]
        mina_s[...] = jnp.where(better, loc_arg, mina_s[...])
        minv_s[...] = jnp.where(better, loc_min, minv_s[...])

    @pl.when(j == kt - 1)
    def _():
        idx_ref[0, 0, :] = mina_s[0, :]


def _dist_body_first(a2_ref, r_ref, cb_ref, b2_ref, idx_ref,
                     rT_s, minv_s, mina_s):
    j = pl.program_id(1)
    kt = pl.num_programs(1)

    @pl.when(j == 0)
    def _():
        rT_s[...] = r_ref[...].T

    _dist_common(rT_s[...], a2_ref[0], cb_ref, b2_ref, idx_ref,
                 minv_s, mina_s, j, kt)


def _dist_body_update(r_ref, q_ref, cb_ref, b2_ref, idx_ref, rout_ref,
                      a2out_ref, rT_s, a2_s, minv_s, mina_s):
    j = pl.program_id(1)
    kt = pl.num_programs(1)

    @pl.when(j == 0)
    def _():
        r = r_ref[...]
        q = q_ref[...]
        qst = r + (q - r)
        rn = r - qst
        rout_ref[...] = rn
        rT_s[...] = rn.T
        a2row = jnp.sum(rn * rn, axis=1)
        a2_s[0, :] = a2row
        a2out_ref[0, 0, :] = a2row

    _dist_common(rT_s[...], a2_s[...], cb_ref, b2_ref, idx_ref,
                 minv_s, mina_s, j, kt)


def _dist_first(xf, a2, cb, b2, nb, k):
    kt = k // _KB
    return pl.pallas_call(
        _dist_body_first,
        grid=(nb, kt),
        in_specs=[
            pl.BlockSpec((1, 1, _TB), lambda i, j: (i, 0, 0)),
            pl.BlockSpec((_TB, _D), lambda i, j: (i, 0)),
            pl.BlockSpec((_KB, _D), lambda i, j: (j, 0)),
            pl.BlockSpec((_KB, 1), lambda i, j: (j, 0)),
        ],
        out_specs=pl.BlockSpec((1, 1, _TB), lambda i, j: (i, 0, 0)),
        out_shape=jax.ShapeDtypeStruct((nb, 1, _TB), jnp.int32),
        scratch_shapes=[
            pltpu.VMEM((_D, _TB), jnp.float32),
            pltpu.VMEM((1, _TB), jnp.float32),
            pltpu.VMEM((1, _TB), jnp.int32),
        ],
    )(a2, xf, cb, b2)


def _dist_update(r, q, cb, b2, nb, k):
    kt = k // _KB
    n = nb * _TB
    return pl.pallas_call(
        _dist_body_update,
        grid=(nb, kt),
        in_specs=[
            pl.BlockSpec((_TB, _D), lambda i, j: (i, 0)),
            pl.BlockSpec((_TB, _D), lambda i, j: (i, 0)),
            pl.BlockSpec((_KB, _D), lambda i, j: (j, 0)),
            pl.BlockSpec((_KB, 1), lambda i, j: (j, 0)),
        ],
        out_specs=[
            pl.BlockSpec((1, 1, _TB), lambda i, j: (i, 0, 0)),
            pl.BlockSpec((_TB, _D), lambda i, j: (i, 0)),
            pl.BlockSpec((1, 1, _TB), lambda i, j: (i, 0, 0)),
        ],
        out_shape=[
            jax.ShapeDtypeStruct((nb, 1, _TB), jnp.int32),
            jax.ShapeDtypeStruct((n, _D), jnp.float32),
            jax.ShapeDtypeStruct((nb, 1, _TB), jnp.float32),
        ],
        scratch_shapes=[
            pltpu.VMEM((_D, _TB), jnp.float32),
            pltpu.VMEM((1, _TB), jnp.float32),
            pltpu.VMEM((1, _TB), jnp.float32),
            pltpu.VMEM((1, _TB), jnp.int32),
        ],
    )(r, q, cb, b2)


def _final_body(x_ref, r_ref, q_ref, quant_ref, a2out_ref):
    x = x_ref[...]
    r = r_ref[...]
    q = q_ref[...]
    qst = r + (q - r)
    rn = r - qst
    quant_ref[...] = x - rn
    a2out_ref[0, 0, :] = jnp.sum(rn * rn, axis=1)


def _final(xf, r, q, nb):
    n = nb * _TB
    return pl.pallas_call(
        _final_body,
        grid=(nb,),
        in_specs=[
            pl.BlockSpec((_TB, _D), lambda i: (i, 0)),
            pl.BlockSpec((_TB, _D), lambda i: (i, 0)),
            pl.BlockSpec((_TB, _D), lambda i: (i, 0)),
        ],
        out_specs=[
            pl.BlockSpec((_TB, _D), lambda i: (i, 0)),
            pl.BlockSpec((1, 1, _TB), lambda i: (i, 0, 0)),
        ],
        out_shape=[
            jax.ShapeDtypeStruct((n, _D), jnp.float32),
            jax.ShapeDtypeStruct((nb, 1, _TB), jnp.float32),
        ],
    )(xf, r, q)


def _make_sc_gather(n_tokens):
    info = plsc.get_sparse_core_info()
    nw = info.num_cores * info.num_subcores
    bpw = n_tokens // nw
    mesh = plsc.VectorSubcoreMesh(core_axis_name="c", subcore_axis_name="s")

    def body(table_hbm, idx_hbm, out_hbm, idx_v, rows_v, sem):
        wid = lax.axis_index("s") * info.num_cores + lax.axis_index("c")
        base = wid * bpw
        pltpu.sync_copy(idx_hbm.at[pl.ds(base, bpw)], idx_v)
        pltpu.async_copy(table_hbm.at[idx_v], rows_v, sem).wait()
        pltpu.sync_copy(rows_v, out_hbm.at[pl.ds(base, bpw)])

    return functools.partial(
        pl.kernel,
        mesh=mesh,
        out_type=jax.ShapeDtypeStruct((n_tokens, _D), jnp.float32),
        scratch_types=[
            pltpu.VMEM((bpw,), jnp.int32),
            pltpu.VMEM((bpw, _D), jnp.float32),
            pltpu.SemaphoreType.DMA,
        ],
    )(body)


def kernel(x, codebook_0, codebook_1, codebook_2, codebook_3):
    codebooks = [codebook_0, codebook_1, codebook_2, codebook_3]
    b, t, d = x.shape
    n = b * t
    nb = n // _TB
    xf = x.reshape(n, d)

    b2s = [jnp.sum(cb * cb, axis=1)[:, None] for cb in codebooks]
    a2_0 = jnp.sum(xf * xf, axis=1).reshape(nb, 1, _TB)

    sc_gather = _make_sc_gather(n)

    idx0 = _dist_first(xf, a2_0, codebooks[0], b2s[0], nb, codebooks[0].shape[0])
    q = sc_gather(codebooks[0], idx0.reshape(n))

    indices = [idx0]
    a2_sums = []
    r = xf
    for l in (1, 2, 3):
        k = codebooks[l].shape[0]
        idx_l, r, a2_l = _dist_update(r, q, codebooks[l], b2s[l], nb, k)
        indices.append(idx_l)
        a2_sums.append(jnp.sum(a2_l))
        q = sc_gather(codebooks[l], idx_l.reshape(n))

    quant, a2_last = _final(xf, r, q, nb)
    a2_sums.append(jnp.sum(a2_last))

    total_commit = jnp.asarray(0.0, dtype=jnp.float32)
    scale = jnp.float32(_COMMIT_W / (n * d))
    for s in a2_sums:
        total_commit = total_commit + s * scale

    quantized = quant.reshape(b, t, d)
    all_indices = jnp.stack([ix.reshape(b, t) for ix in indices], axis=-1)
    return quantized, all_indices, total_commit


# speedup vs baseline: 1.9032x; 1.5108x over previous
import functools

import jax
import jax.numpy as jnp
from jax import lax
from jax.experimental import pallas as pl
from jax.experimental.pallas import tpu as pltpu
from jax.experimental.pallas import tpu_sc as plsc

_D = 768
_TB = 256
_CH = 1024
_COMMIT_W = 0.25


def _argmin_full(rT, a2v, cb_ref, b2_ref, idx_ref):
    k = cb_ref.shape[0]
    big = jnp.int32(2 ** 30)
    best_v = None
    best_a = None
    for c in range(k // _CH):
        cb = cb_ref[c * _CH:(c + 1) * _CH, :]
        b2c = b2_ref[c * _CH:(c + 1) * _CH, :]
        ab = lax.dot_general(cb, rT, (((1,), (0,)), ((), ())))
        d2 = jnp.maximum((b2c + a2v) - 2.0 * ab, 0.0)
        lm = jnp.min(d2, axis=0, keepdims=True)
        ii = lax.broadcasted_iota(jnp.int32, d2.shape, 0) + c * _CH
        la = jnp.min(jnp.where(d2 == lm, ii, big), axis=0, keepdims=True)
        if best_v is None:
            best_v, best_a = lm, la
        else:
            better = lm < best_v
            best_a = jnp.where(better, la, best_a)
            best_v = jnp.where(better, lm, best_v)
    idx_ref[0, 0, :] = best_a[0]


def _dist_body_first(a2_ref, r_ref, cb_ref, b2_ref, idx_ref, rT_s):
    rT_s[...] = r_ref[...].T
    _argmin_full(rT_s[...], a2_ref[0], cb_ref, b2_ref, idx_ref)


def _dist_body_update(r_ref, q_ref, cb_ref, b2_ref, idx_ref, rout_ref,
                      a2out_ref, rT_s):
    r = r_ref[...]
    q = q_ref[...]
    qst = r + (q - r)
    rn = r - qst
    rout_ref[...] = rn
    rT_s[...] = rn.T
    a2row = jnp.sum(rn * rn, axis=1)
    a2out_ref[0, 0, :] = a2row
    _argmin_full(rT_s[...], a2row[None, :], cb_ref, b2_ref, idx_ref)


def _dist_first(xf, a2, cb, b2, nb, k):
    return pl.pallas_call(
        _dist_body_first,
        grid=(nb,),
        in_specs=[
            pl.BlockSpec((1, 1, _TB), lambda i: (i, 0, 0)),
            pl.BlockSpec((_TB, _D), lambda i: (i, 0)),
            pl.BlockSpec((k, _D), lambda i: (0, 0)),
            pl.BlockSpec((k, 1), lambda i: (0, 0)),
        ],
        out_specs=pl.BlockSpec((1, 1, _TB), lambda i: (i, 0, 0)),
        out_shape=jax.ShapeDtypeStruct((nb, 1, _TB), jnp.int32),
        scratch_shapes=[
            pltpu.VMEM((_D, _TB), jnp.float32),
        ],
    )(a2, xf, cb, b2)


def _dist_update(r, q, cb, b2, nb, k):
    n = nb * _TB
    return pl.pallas_call(
        _dist_body_update,
        grid=(nb,),
        in_specs=[
            pl.BlockSpec((_TB, _D), lambda i: (i, 0)),
            pl.BlockSpec((_TB, _D), lambda i: (i, 0)),
            pl.BlockSpec((k, _D), lambda i: (0, 0)),
            pl.BlockSpec((k, 1), lambda i: (0, 0)),
        ],
        out_specs=[
            pl.BlockSpec((1, 1, _TB), lambda i: (i, 0, 0)),
            pl.BlockSpec((_TB, _D), lambda i: (i, 0)),
            pl.BlockSpec((1, 1, _TB), lambda i: (i, 0, 0)),
        ],
        out_shape=[
            jax.ShapeDtypeStruct((nb, 1, _TB), jnp.int32),
            jax.ShapeDtypeStruct((n, _D), jnp.float32),
            jax.ShapeDtypeStruct((nb, 1, _TB), jnp.float32),
        ],
        scratch_shapes=[
            pltpu.VMEM((_D, _TB), jnp.float32),
        ],
    )(r, q, cb, b2)


def _final_body(x_ref, r_ref, q_ref, quant_ref, a2out_ref):
    x = x_ref[...]
    r = r_ref[...]
    q = q_ref[...]
    qst = r + (q - r)
    rn = r - qst
    quant_ref[...] = x - rn
    a2out_ref[0, 0, :] = jnp.sum(rn * rn, axis=1)


def _final(xf, r, q, nb):
    n = nb * _TB
    return pl.pallas_call(
        _final_body,
        grid=(nb,),
        in_specs=[
            pl.BlockSpec((_TB, _D), lambda i: (i, 0)),
            pl.BlockSpec((_TB, _D), lambda i: (i, 0)),
            pl.BlockSpec((_TB, _D), lambda i: (i, 0)),
        ],
        out_specs=[
            pl.BlockSpec((_TB, _D), lambda i: (i, 0)),
            pl.BlockSpec((1, 1, _TB), lambda i: (i, 0, 0)),
        ],
        out_shape=[
            jax.ShapeDtypeStruct((n, _D), jnp.float32),
            jax.ShapeDtypeStruct((nb, 1, _TB), jnp.float32),
        ],
    )(xf, r, q)


def _make_sc_gather(n_tokens):
    info = plsc.get_sparse_core_info()
    nw = info.num_cores * info.num_subcores
    bpw = n_tokens // nw
    mesh = plsc.VectorSubcoreMesh(core_axis_name="c", subcore_axis_name="s")

    def body(table_hbm, idx_hbm, out_hbm, idx_v, rows_v, sem):
        wid = lax.axis_index("s") * info.num_cores + lax.axis_index("c")
        base = wid * bpw
        pltpu.sync_copy(idx_hbm.at[pl.ds(base, bpw)], idx_v)
        pltpu.async_copy(table_hbm.at[idx_v], rows_v, sem).wait()
        pltpu.sync_copy(rows_v, out_hbm.at[pl.ds(base, bpw)])

    return functools.partial(
        pl.kernel,
        mesh=mesh,
        out_type=jax.ShapeDtypeStruct((n_tokens, _D), jnp.float32),
        scratch_types=[
            pltpu.VMEM((bpw,), jnp.int32),
            pltpu.VMEM((bpw, _D), jnp.float32),
            pltpu.SemaphoreType.DMA,
        ],
    )(body)


def kernel(x, codebook_0, codebook_1, codebook_2, codebook_3):
    codebooks = [codebook_0, codebook_1, codebook_2, codebook_3]
    b, t, d = x.shape
    n = b * t
    nb = n // _TB
    xf = x.reshape(n, d)

    b2s = [jnp.sum(cb * cb, axis=1)[:, None] for cb in codebooks]
    a2_0 = jnp.sum(xf * xf, axis=1).reshape(nb, 1, _TB)

    sc_gather = _make_sc_gather(n)

    idx0 = _dist_first(xf, a2_0, codebooks[0], b2s[0], nb, codebooks[0].shape[0])
    q = sc_gather(codebooks[0], idx0.reshape(n))

    indices = [idx0]
    a2_sums = []
    r = xf
    for l in (1, 2, 3):
        k = codebooks[l].shape[0]
        idx_l, r, a2_l = _dist_update(r, q, codebooks[l], b2s[l], nb, k)
        indices.append(idx_l)
        a2_sums.append(jnp.sum(a2_l))
        q = sc_gather(codebooks[l], idx_l.reshape(n))

    quant, a2_last = _final(xf, r, q, nb)
    a2_sums.append(jnp.sum(a2_last))

    total_commit = jnp.asarray(0.0, dtype=jnp.float32)
    scale = jnp.float32(_COMMIT_W / (n * d))
    for s in a2_sums:
        total_commit = total_commit + s * scale

    quantized = quant.reshape(b, t, d)
    all_indices = jnp.stack([ix.reshape(b, t) for ix in indices], axis=-1)
    return quantized, all_indices, total_commit


# speedup vs baseline: 2.0437x; 1.0738x over previous
import functools

import jax
import jax.numpy as jnp
from jax import lax
from jax.experimental import pallas as pl
from jax.experimental.pallas import tpu as pltpu
from jax.experimental.pallas import tpu_sc as plsc

_D = 768
_TB = 256
_CH = 1024
_COMMIT_W = 0.25


def _compute_b2(cb_ref, b2_s):
    k = cb_ref.shape[0]
    for c in range(k // _CH):
        cb = cb_ref[c * _CH:(c + 1) * _CH, :]
        b2_s[c * _CH:(c + 1) * _CH, :] = jnp.sum(cb * cb, axis=1,
                                                 keepdims=True)


def _argmin_full(rT, a2v, cb_ref, b2_s, idx_ref):
    k = cb_ref.shape[0]
    big = jnp.int32(2 ** 30)
    best_v = None
    best_a = None
    for c in range(k // _CH):
        cb = cb_ref[c * _CH:(c + 1) * _CH, :]
        b2c = b2_s[c * _CH:(c + 1) * _CH, :]
        ab = lax.dot_general(cb, rT, (((1,), (0,)), ((), ())))
        d2 = jnp.maximum((b2c + a2v) - 2.0 * ab, 0.0)
        lm = jnp.min(d2, axis=0, keepdims=True)
        ii = lax.broadcasted_iota(jnp.int32, d2.shape, 0) + c * _CH
        la = jnp.min(jnp.where(d2 == lm, ii, big), axis=0, keepdims=True)
        if best_v is None:
            best_v, best_a = lm, la
        else:
            better = lm < best_v
            best_a = jnp.where(better, la, best_a)
            best_v = jnp.where(better, lm, best_v)
    idx_ref[0, 0, :] = best_a[0]


def _dist_body_first(a2_ref, r_ref, cb_ref, idx_ref, rT_s, b2_s):
    @pl.when(pl.program_id(0) == 0)
    def _():
        _compute_b2(cb_ref, b2_s)

    rT_s[...] = r_ref[...].T
    _argmin_full(rT_s[...], a2_ref[0], cb_ref, b2_s, idx_ref)


def _dist_body_update(r_ref, q_ref, cb_ref, idx_ref, rout_ref,
                      a2out_ref, rT_s, b2_s):
    @pl.when(pl.program_id(0) == 0)
    def _():
        _compute_b2(cb_ref, b2_s)

    r = r_ref[...]
    q = q_ref[...]
    qst = r + (q - r)
    rn = r - qst
    rout_ref[...] = rn
    rT_s[...] = rn.T
    a2row = jnp.sum(rn * rn, axis=1)
    a2out_ref[0, 0, :] = a2row
    _argmin_full(rT_s[...], a2row[None, :], cb_ref, b2_s, idx_ref)


def _dist_first(xf, a2, cb, nb, k):
    return pl.pallas_call(
        _dist_body_first,
        grid=(nb,),
        in_specs=[
            pl.BlockSpec((1, 1, _TB), lambda i: (i, 0, 0)),
            pl.BlockSpec((_TB, _D), lambda i: (i, 0)),
            pl.BlockSpec((k, _D), lambda i: (0, 0)),
        ],
        out_specs=pl.BlockSpec((1, 1, _TB), lambda i: (i, 0, 0)),
        out_shape=jax.ShapeDtypeStruct((nb, 1, _TB), jnp.int32),
        scratch_shapes=[
            pltpu.VMEM((_D, _TB), jnp.float32),
            pltpu.VMEM((k, 1), jnp.float32),
        ],
    )(a2, xf, cb)


def _dist_update(r, q, cb, nb, k):
    n = nb * _TB
    return pl.pallas_call(
        _dist_body_update,
        grid=(nb,),
        in_specs=[
            pl.BlockSpec((_TB, _D), lambda i: (i, 0)),
            pl.BlockSpec((_TB, _D), lambda i: (i, 0)),
            pl.BlockSpec((k, _D), lambda i: (0, 0)),
        ],
        out_specs=[
            pl.BlockSpec((1, 1, _TB), lambda i: (i, 0, 0)),
            pl.BlockSpec((_TB, _D), lambda i: (i, 0)),
            pl.BlockSpec((1, 1, _TB), lambda i: (i, 0, 0)),
        ],
        out_shape=[
            jax.ShapeDtypeStruct((nb, 1, _TB), jnp.int32),
            jax.ShapeDtypeStruct((n, _D), jnp.float32),
            jax.ShapeDtypeStruct((nb, 1, _TB), jnp.float32),
        ],
        scratch_shapes=[
            pltpu.VMEM((_D, _TB), jnp.float32),
            pltpu.VMEM((k, 1), jnp.float32),
        ],
    )(r, q, cb)


def _final_body(x_ref, r_ref, q_ref, quant_ref, a2out_ref):
    x = x_ref[...]
    r = r_ref[...]
    q = q_ref[...]
    qst = r + (q - r)
    rn = r - qst
    quant_ref[...] = x - rn
    a2out_ref[0, 0, :] = jnp.sum(rn * rn, axis=1)


def _final(xf, r, q, nb):
    n = nb * _TB
    return pl.pallas_call(
        _final_body,
        grid=(nb,),
        in_specs=[
            pl.BlockSpec((_TB, _D), lambda i: (i, 0)),
            pl.BlockSpec((_TB, _D), lambda i: (i, 0)),
            pl.BlockSpec((_TB, _D), lambda i: (i, 0)),
        ],
        out_specs=[
            pl.BlockSpec((_TB, _D), lambda i: (i, 0)),
            pl.BlockSpec((1, 1, _TB), lambda i: (i, 0, 0)),
        ],
        out_shape=[
            jax.ShapeDtypeStruct((n, _D), jnp.float32),
            jax.ShapeDtypeStruct((nb, 1, _TB), jnp.float32),
        ],
    )(xf, r, q)


def _make_sc_gather(n_tokens):
    info = plsc.get_sparse_core_info()
    nw = info.num_cores * info.num_subcores
    bpw = n_tokens // nw
    mesh = plsc.VectorSubcoreMesh(core_axis_name="c", subcore_axis_name="s")

    def body(table_hbm, idx_hbm, out_hbm, idx_v, rows_v, sem):
        wid = lax.axis_index("s") * info.num_cores + lax.axis_index("c")
        base = wid * bpw
        pltpu.sync_copy(idx_hbm.at[pl.ds(base, bpw)], idx_v)
        pltpu.async_copy(table_hbm.at[idx_v], rows_v, sem).wait()
        pltpu.sync_copy(rows_v, out_hbm.at[pl.ds(base, bpw)])

    return functools.partial(
        pl.kernel,
        mesh=mesh,
        out_type=jax.ShapeDtypeStruct((n_tokens, _D), jnp.float32),
        scratch_types=[
            pltpu.VMEM((bpw,), jnp.int32),
            pltpu.VMEM((bpw, _D), jnp.float32),
            pltpu.SemaphoreType.DMA,
        ],
    )(body)


def kernel(x, codebook_0, codebook_1, codebook_2, codebook_3):
    codebooks = [codebook_0, codebook_1, codebook_2, codebook_3]
    b, t, d = x.shape
    n = b * t
    nb = n // _TB
    xf = x.reshape(n, d)

    a2_0 = jnp.sum(xf * xf, axis=1).reshape(nb, 1, _TB)

    sc_gather = _make_sc_gather(n)

    idx0 = _dist_first(xf, a2_0, codebooks[0], nb, codebooks[0].shape[0])
    q = sc_gather(codebooks[0], idx0.reshape(n))

    indices = [idx0]
    a2_sums = []
    r = xf
    for l in (1, 2, 3):
        k = codebooks[l].shape[0]
        idx_l, r, a2_l = _dist_update(r, q, codebooks[l], nb, k)
        indices.append(idx_l)
        a2_sums.append(jnp.sum(a2_l))
        q = sc_gather(codebooks[l], idx_l.reshape(n))

    quant, a2_last = _final(xf, r, q, nb)
    a2_sums.append(jnp.sum(a2_last))

    total_commit = jnp.asarray(0.0, dtype=jnp.float32)
    scale = jnp.float32(_COMMIT_W / (n * d))
    for s in a2_sums:
        total_commit = total_commit + s * scale

    quantized = quant.reshape(b, t, d)
    all_indices = jnp.stack([ix.reshape(b, t) for ix in indices], axis=-1)
    return quantized, all_indices, total_commit
